# baseline (speedup 1.0000x reference)
import functools

import jax
import jax.numpy as jnp
from jax.experimental import pallas as pl
from jax.experimental.pallas import tpu as pltpu

VAR0 = 0.1
VAR1 = 0.2
THR = 0.5
NEG_POS = 3.0
BISECT_ITERS = 24
EXP_CLAMP = 80.0


def _match_body(pr_ref, tl_ref, th_ref, lab_ref, loc_ref,
                conf_ref, lloc_ref, npos_ref, *, n_obj, n_pri):
    pc = pr_ref[0, 0:1, :]
    pw = pr_ref[0, 1:2, :]
    plo = pc - pw * 0.5
    phi = pc + pw * 0.5
    tl = tl_ref[0]
    th = th_ref[0]
    lab = lab_ref[0]

    lo = jnp.maximum(plo, tl)
    hi = jnp.minimum(phi, th)
    inter = jnp.maximum(hi - lo, 0.0)
    ov = inter / ((th - tl) + pw - inter)

    iota_s = jax.lax.broadcasted_iota(jnp.int32, (n_obj, n_pri), 0)
    iota_l = jax.lax.broadcasted_iota(jnp.int32, (n_obj, n_pri), 1)
    bto = jnp.max(ov, axis=0, keepdims=True)
    bti = jnp.min(jnp.where(ov == bto, iota_s, n_obj), axis=0,
                  keepdims=True)
    mt = jnp.max(ov, axis=1, keepdims=True)
    tbest = jnp.min(jnp.where(ov == mt, iota_l, n_pri), axis=1,
                    keepdims=True)
    fm = iota_l == tbest
    jmax = jnp.max(jnp.where(fm, iota_s, -1), axis=0, keepdims=True)
    forced = jmax >= 0
    bti = jnp.where(forced, jmax, bti)
    bto = jnp.where(forced, 2.0, bto)
    sel = iota_s == bti
    mlo = jnp.sum(jnp.where(sel, tl, 0.0), axis=0, keepdims=True)
    mhi = jnp.sum(jnp.where(sel, th, 0.0), axis=0, keepdims=True)
    mlab = jnp.sum(jnp.where(sel, lab, 0.0), axis=0, keepdims=True)
    conf = jnp.where(bto < THR, 0.0, mlab + 1.0)
    pos = conf > 0.0
    gc = ((mlo + mhi) * 0.5 - pc) / (VAR0 * pw)
    gw = jnp.log((mhi - mlo) / pw) / VAR1
    dc = loc_ref[0, 0:1, :] - gc
    dw = loc_ref[0, 1:2, :] - gw
    adc = jnp.abs(dc)
    adw = jnp.abs(dw)
    sl1 = (jnp.where(adc < 1.0, 0.5 * dc * dc, adc - 0.5)
           + jnp.where(adw < 1.0, 0.5 * dw * dw, adw - 0.5))
    lloc_ref[0] = jnp.sum(jnp.where(pos, sl1, 0.0)).reshape(1, 1)
    npos_ref[0] = jnp.sum(jnp.where(pos, 1.0, 0.0)).reshape(1, 1)
    conf_ref[0] = conf


def _ce_body(conf_ref, ct_ref, lc_ref, spce_ref):
    v = conf_ref[0]
    ct = ct_ref[0]
    s = jnp.sum(jnp.exp(jnp.minimum(v, EXP_CLAMP)), axis=0, keepdims=True)
    lse = jnp.log(s)
    tgt = ct.astype(jnp.int32)
    iota_s = jax.lax.broadcasted_iota(jnp.int32, v.shape, 0)
    ctgt = jnp.sum(jnp.where(iota_s == tgt, v, 0.0), axis=0, keepdims=True)
    ce = lse - ctgt
    pos = ct > 0.0
    lc_ref[0] = jnp.where(pos, 0.0, ce)
    spce_ref[0] = jnp.sum(jnp.where(pos, ce, 0.0)).reshape(1, 1)


def _mine_body(lc0_ref, lc1_ref, spce_ref, npos_ref, lloc_ref, out_ref,
               *, n_batch, n_pri):
    vv = jnp.concatenate([lc0_ref[0], lc1_ref[0]], axis=0)
    spce = spce_ref[0]
    npos = npos_ref[0]
    k = jnp.minimum(npos * NEG_POS, float(n_pri - 1))
    cnt0 = jnp.sum(jnp.where(vv > 0.0, 1.0, 0.0), axis=1, keepdims=True)
    k = jnp.minimum(k, cnt0)
    lo = jnp.zeros((n_batch, 1), jnp.float32)
    hi = jnp.max(vv, axis=1, keepdims=True)

    def body(_, lohi):
        lo, hi = lohi
        mid = 0.5 * (lo + hi)
        cnt = jnp.sum(jnp.where(vv > mid, 1.0, 0.0), axis=1, keepdims=True)
        pred = cnt > k
        return jnp.where(pred, mid, lo), jnp.where(pred, hi, mid)

    lo, hi = jax.lax.fori_loop(0, BISECT_ITERS, body, (lo, hi))
    above = vv > hi
    cnt_hi = jnp.sum(jnp.where(above, 1.0, 0.0), axis=1, keepdims=True)
    sum_hi = jnp.sum(jnp.where(above, vv, 0.0), axis=1, keepdims=True)
    topk = sum_hi + hi * jnp.maximum(k - cnt_hi, 0.0)
    n_tot = jnp.sum(npos)
    a = (jnp.sum(lloc_ref[0]) / n_tot).reshape(1, 1)
    b2 = ((jnp.sum(spce) + jnp.sum(topk)) / n_tot).reshape(1, 1)
    out_ref[0] = jnp.concatenate([a, b2], axis=1)


def _forward(loc_data, conf_data, priors, targets, interpret=False):
    B, P, _ = loc_data.shape
    C = conf_data.shape[2]
    O = targets.shape[1]

    prT = priors.T.reshape(1, 2, P)
    locT = jnp.swapaxes(loc_data, 1, 2)
    tl = targets[:, :, 0:1]
    th = targets[:, :, 1:2]
    lab = targets[:, :, 2:3]

    f32 = jnp.float32
    conf_t, lloc, npos = pl.pallas_call(
        functools.partial(_match_body, n_obj=O, n_pri=P),
        grid=(B,),
        in_specs=[
            pl.BlockSpec((1, 2, P), lambda b: (0, 0, 0)),
            pl.BlockSpec((1, O, 1), lambda b: (b, 0, 0)),
            pl.BlockSpec((1, O, 1), lambda b: (b, 0, 0)),
            pl.BlockSpec((1, O, 1), lambda b: (b, 0, 0)),
            pl.BlockSpec((1, 2, P), lambda b: (b, 0, 0)),
        ],
        out_specs=[
            pl.BlockSpec((1, 1, P), lambda b: (b, 0, 0)),
            pl.BlockSpec((1, 1, 1), lambda b: (b, 0, 0)),
            pl.BlockSpec((1, 1, 1), lambda b: (b, 0, 0)),
        ],
        out_shape=[
            jax.ShapeDtypeStruct((B, 1, P), f32),
            jax.ShapeDtypeStruct((B, 1, 1), f32),
            jax.ShapeDtypeStruct((B, 1, 1), f32),
        ],
        interpret=interpret,
    )(prT, tl, th, lab, locT)

    H = B // 2
    lcs, spces = [], []
    for h in range(2):
        confT_h = jnp.swapaxes(conf_data[h * H:(h + 1) * H], 1, 2)
        ct_h = conf_t[h * H:(h + 1) * H]
        lc_h, spce_h = pl.pallas_call(
            _ce_body,
            grid=(H,),
            in_specs=[
                pl.BlockSpec((1, C, P), lambda b: (b, 0, 0)),
                pl.BlockSpec((1, 1, P), lambda b: (b, 0, 0)),
            ],
            out_specs=[
                pl.BlockSpec((1, 1, P), lambda b: (b, 0, 0)),
                pl.BlockSpec((1, 1, 1), lambda b: (b, 0, 0)),
            ],
            out_shape=[
                jax.ShapeDtypeStruct((H, 1, P), f32),
                jax.ShapeDtypeStruct((H, 1, 1), f32),
            ],
            interpret=interpret,
        )(confT_h, ct_h)
        lcs.append(lc_h)
        spces.append(spce_h)
    spce = jnp.concatenate(spces, axis=0)

    out = pl.pallas_call(
        functools.partial(_mine_body, n_batch=B, n_pri=P),
        grid=(1,),
        in_specs=[
            pl.BlockSpec((1, H, P), lambda i: (0, 0, 0)),
            pl.BlockSpec((1, H, P), lambda i: (0, 0, 0)),
            pl.BlockSpec((1, B, 1), lambda i: (0, 0, 0)),
            pl.BlockSpec((1, B, 1), lambda i: (0, 0, 0)),
            pl.BlockSpec((1, B, 1), lambda i: (0, 0, 0)),
        ],
        out_specs=pl.BlockSpec((1, 1, 2), lambda i: (0, 0, 0)),
        out_shape=jax.ShapeDtypeStruct((1, 1, 2), f32),
        interpret=interpret,
    )(lcs[0].reshape(1, H, P), lcs[1].reshape(1, H, P),
      spce.reshape(1, B, 1), npos.reshape(1, B, 1), lloc.reshape(1, B, 1))

    return out[0, 0, 0], out[0, 0, 1]


def kernel(loc_data, conf_data, priors, targets):
    return _forward(loc_data, conf_data, priors, targets)


# speedup vs baseline: 1.1733x; 1.1733x over previous
import functools

import jax
import jax.numpy as jnp
from jax.experimental import pallas as pl
from jax.experimental.pallas import tpu as pltpu

VAR0 = 0.1
VAR1 = 0.2
THR = 0.5
NEG_POS = 3.0
BISECT_ITERS = 20
EXP_CLAMP = 80.0


def _match_body(pr_ref, tl_ref, th_ref, lab_ref, loc_ref,
                conf_ref, lloc_ref, npos_ref, *, n_obj, n_pri):
    pc = pr_ref[0, 0:1, :]
    pw = pr_ref[0, 1:2, :]
    plo = pc - pw * 0.5
    phi = pc + pw * 0.5
    tl = tl_ref[0]
    th = th_ref[0]
    lab = lab_ref[0]

    lo = jnp.maximum(plo, tl)
    hi = jnp.minimum(phi, th)
    inter = jnp.maximum(hi - lo, 0.0)
    ov = inter / ((th - tl) + pw - inter)

    iota_s = jax.lax.broadcasted_iota(jnp.int32, (n_obj, n_pri), 0)
    iota_l = jax.lax.broadcasted_iota(jnp.int32, (n_obj, n_pri), 1)
    bto = jnp.max(ov, axis=0, keepdims=True)
    bti = jnp.min(jnp.where(ov == bto, iota_s, n_obj), axis=0,
                  keepdims=True)
    mt = jnp.max(ov, axis=1, keepdims=True)
    tbest = jnp.min(jnp.where(ov == mt, iota_l, n_pri), axis=1,
                    keepdims=True)
    fm = iota_l == tbest
    jmax = jnp.max(jnp.where(fm, iota_s, -1), axis=0, keepdims=True)
    forced = jmax >= 0
    bti = jnp.where(forced, jmax, bti)
    bto = jnp.where(forced, 2.0, bto)
    sel = iota_s == bti
    mlo = jnp.sum(jnp.where(sel, tl, 0.0), axis=0, keepdims=True)
    mhi = jnp.sum(jnp.where(sel, th, 0.0), axis=0, keepdims=True)
    mlab = jnp.sum(jnp.where(sel, lab, 0.0), axis=0, keepdims=True)
    conf = jnp.where(bto < THR, 0.0, mlab + 1.0)
    pos = conf > 0.0
    gc = ((mlo + mhi) * 0.5 - pc) / (VAR0 * pw)
    gw = jnp.log((mhi - mlo) / pw) / VAR1
    dc = loc_ref[0, 0:1, :] - gc
    dw = loc_ref[0, 1:2, :] - gw
    adc = jnp.abs(dc)
    adw = jnp.abs(dw)
    sl1 = (jnp.where(adc < 1.0, 0.5 * dc * dc, adc - 0.5)
           + jnp.where(adw < 1.0, 0.5 * dw * dw, adw - 0.5))
    lloc_ref[0] = jnp.sum(jnp.where(pos, sl1, 0.0)).reshape(1, 1)
    npos_ref[0] = jnp.sum(jnp.where(pos, 1.0, 0.0)).reshape(1, 1)
    conf_ref[0] = conf


def _ce_mine_body(conf_ref, ct_ref, npos_ref, lloc_ref, out_ref,
                  lcs_ref, spc_ref, *, n_batch, n_pri):
    b = pl.program_id(0)
    v = conf_ref[0]
    ct = ct_ref[0]
    s = jnp.sum(jnp.exp(jnp.minimum(v, EXP_CLAMP)), axis=0, keepdims=True)
    lse = jnp.log(s)
    tgt = ct.astype(jnp.int32)
    iota_s = jax.lax.broadcasted_iota(jnp.int32, v.shape, 0)
    ctgt = jnp.sum(jnp.where(iota_s == tgt, v, 0.0), axis=0, keepdims=True)
    ce = lse - ctgt
    pos = ct > 0.0
    lcs_ref[b] = jnp.where(pos, 0.0, ce)
    spc_ref[b] = jnp.sum(jnp.where(pos, ce, 0.0)).reshape(1, 1)

    @pl.when(b == n_batch - 1)
    def _():
        vv = lcs_ref[:, 0, :]
        spce = spc_ref[:, 0, :]
        npos = npos_ref[0]
        k = jnp.minimum(npos * NEG_POS, float(n_pri - 1))
        cnt0 = jnp.sum(jnp.where(vv > 0.0, 1.0, 0.0), axis=1, keepdims=True)
        k = jnp.minimum(k, cnt0)
        lo = jnp.zeros((n_batch, 1), jnp.float32)
        hi = jnp.max(vv, axis=1, keepdims=True)

        def body(_, lohi):
            lo, hi = lohi
            mid = 0.5 * (lo + hi)
            cnt = jnp.sum(jnp.where(vv > mid, 1.0, 0.0), axis=1,
                          keepdims=True)
            pred = cnt > k
            return jnp.where(pred, mid, lo), jnp.where(pred, hi, mid)

        lo, hi = jax.lax.fori_loop(0, BISECT_ITERS, body, (lo, hi))
        above = vv > hi
        cnt_hi = jnp.sum(jnp.where(above, 1.0, 0.0), axis=1, keepdims=True)
        sum_hi = jnp.sum(jnp.where(above, vv, 0.0), axis=1, keepdims=True)
        topk = sum_hi + hi * jnp.maximum(k - cnt_hi, 0.0)
        n_tot = jnp.sum(npos)
        a = (jnp.sum(lloc_ref[0]) / n_tot).reshape(1, 1)
        b2 = ((jnp.sum(spce) + jnp.sum(topk)) / n_tot).reshape(1, 1)
        out_ref[0] = jnp.concatenate([a, b2], axis=1)


def _forward(loc_data, conf_data, priors, targets, interpret=False):
    B, P, _ = loc_data.shape
    C = conf_data.shape[2]
    O = targets.shape[1]

    prT = priors.T.reshape(1, 2, P)
    confT = jnp.swapaxes(conf_data, 1, 2)
    locT = jnp.swapaxes(loc_data, 1, 2)
    tl = targets[:, :, 0:1]
    th = targets[:, :, 1:2]
    lab = targets[:, :, 2:3]

    f32 = jnp.float32
    conf_t, lloc, npos = pl.pallas_call(
        functools.partial(_match_body, n_obj=O, n_pri=P),
        grid=(B,),
        in_specs=[
            pl.BlockSpec((1, 2, P), lambda b: (0, 0, 0)),
            pl.BlockSpec((1, O, 1), lambda b: (b, 0, 0)),
            pl.BlockSpec((1, O, 1), lambda b: (b, 0, 0)),
            pl.BlockSpec((1, O, 1), lambda b: (b, 0, 0)),
            pl.BlockSpec((1, 2, P), lambda b: (b, 0, 0)),
        ],
        out_specs=[
            pl.BlockSpec((1, 1, P), lambda b: (b, 0, 0)),
            pl.BlockSpec((1, 1, 1), lambda b: (b, 0, 0)),
            pl.BlockSpec((1, 1, 1), lambda b: (b, 0, 0)),
        ],
        out_shape=[
            jax.ShapeDtypeStruct((B, 1, P), f32),
            jax.ShapeDtypeStruct((B, 1, 1), f32),
            jax.ShapeDtypeStruct((B, 1, 1), f32),
        ],
        interpret=interpret,
    )(prT, tl, th, lab, locT)

    out = pl.pallas_call(
        functools.partial(_ce_mine_body, n_batch=B, n_pri=P),
        grid=(B,),
        in_specs=[
            pl.BlockSpec((1, C, P), lambda b: (b, 0, 0)),
            pl.BlockSpec((1, 1, P), lambda b: (b, 0, 0)),
            pl.BlockSpec((1, B, 1), lambda b: (0, 0, 0)),
            pl.BlockSpec((1, B, 1), lambda b: (0, 0, 0)),
        ],
        out_specs=pl.BlockSpec((1, 1, 2), lambda b: (0, 0, 0)),
        out_shape=jax.ShapeDtypeStruct((1, 1, 2), f32),
        scratch_shapes=[
            pltpu.VMEM((B, 1, P), f32),
            pltpu.VMEM((B, 1, 1), f32),
        ],
        interpret=interpret,
    )(confT, conf_t, npos.reshape(1, B, 1), lloc.reshape(1, B, 1))

    return out[0, 0, 0], out[0, 0, 1]


def kernel(loc_data, conf_data, priors, targets):
    return _forward(loc_data, conf_data, priors, targets)
